# baseline (speedup 1.0000x reference)
import functools

import jax
import jax.numpy as jnp
from jax import lax
from jax.experimental import pallas as pl
from jax.experimental.pallas import tpu as pltpu
from jax.experimental.pallas import tpu_sc as plsc

NUM_EMB = 1000000
DIM = 32
BATCH = 16384

NUM_CORES = 2
NUM_SUBCORES = 16
NUM_WORKERS = NUM_CORES * NUM_SUBCORES
B_PER_W = BATCH // NUM_WORKERS
CHUNK = 128
NCHUNK = B_PER_W // CHUNK

_mesh = plsc.VectorSubcoreMesh(core_axis_name="c", subcore_axis_name="s")


@functools.partial(
    pl.kernel,
    mesh=_mesh,
    out_type=jax.ShapeDtypeStruct((BATCH, DIM), jnp.float32),
    scratch_types=[
        pltpu.VMEM((B_PER_W,), jnp.int32),
        pltpu.VMEM((B_PER_W, DIM), jnp.float32),
        pltpu.SemaphoreType.DMA,
    ],
    compiler_params=pltpu.CompilerParams(use_tc_tiling_on_sc=False),
)
def _gather_kernel(t_hbm, table_hbm, out_hbm, idx_v, rows_v, sem):
    wid = lax.axis_index("s") * NUM_CORES + lax.axis_index("c")
    base = wid * B_PER_W
    pltpu.sync_copy(t_hbm.at[pl.ds(base, B_PER_W)], idx_v)
    copies = [
        pltpu.async_copy(
            table_hbm.at[idx_v.at[pl.ds(j * CHUNK, CHUNK)]],
            rows_v.at[pl.ds(j * CHUNK, CHUNK)],
            sem,
        )
        for j in range(NCHUNK)
    ]
    for c in copies:
        c.wait()
    pltpu.sync_copy(rows_v, out_hbm.at[pl.ds(base, B_PER_W)])


def kernel(t, table):
    return _gather_kernel(t.astype(jnp.int32), table)


# speedup vs baseline: 2.2891x; 2.2891x over previous
import functools

import jax
import jax.numpy as jnp
from jax import lax
from jax.experimental import pallas as pl
from jax.experimental.pallas import tpu as pltpu
from jax.experimental.pallas import tpu_sc as plsc

NUM_EMB = 1000000
DIM = 32
BATCH = 16384
SUB = 8
NTILE = NUM_EMB // SUB

NUM_CORES = 2
NUM_SUBCORES = 16
NUM_WORKERS = NUM_CORES * NUM_SUBCORES
B_PER_W = BATCH // NUM_WORKERS
CHUNK = 16
NCHUNK = B_PER_W // CHUNK

_mesh = plsc.VectorSubcoreMesh(core_axis_name="c", subcore_axis_name="s")


@functools.partial(
    pl.kernel,
    mesh=_mesh,
    out_type=jax.ShapeDtypeStruct((BATCH, DIM), jnp.float32),
    scratch_types=[
        pltpu.VMEM((B_PER_W,), jnp.int32),
        pltpu.VMEM((B_PER_W,), jnp.int32),
        pltpu.VMEM((CHUNK, SUB, DIM), jnp.float32),
        pltpu.VMEM((CHUNK, SUB, DIM), jnp.float32),
        pltpu.VMEM((B_PER_W, DIM), jnp.float32),
        pltpu.SemaphoreType.DMA,
        pltpu.SemaphoreType.DMA,
    ],
)
def _gather_kernel(
    t_hbm, tiles_hbm, out_hbm, off_v, row_v, wide_a, wide_b, out_v, sem, sem2
):
    wid = lax.axis_index("s") * NUM_CORES + lax.axis_index("c")
    base = wid * B_PER_W
    pltpu.async_copy(t_hbm.at[pl.ds(base, B_PER_W)], row_v, sem).wait()

    def widen(v, _):
        m16 = row_v[pl.ds(v * 16, 16)]
        off_v[pl.ds(v * 16, 16)] = m16 & (SUB - 1)
        row_v[pl.ds(v * 16, 16)] = lax.shift_right_logical(m16, 3)
        return 0

    lax.fori_loop(0, B_PER_W // 16, widen, 0)

    bufs = [wide_a, wide_b]
    sems = [sem, sem2]

    def start_gather(j, buf):
        def issue(v, _):
            t16 = row_v[pl.ds(j * CHUNK + v * 16, 16)]
            for lane in range(16):
                pltpu.async_copy(
                    tiles_hbm.at[t16[lane]],
                    buf.at[v * 16 + lane],
                    sems[j % 2],
                )
            return 0

        lax.fori_loop(0, CHUNK // 16, issue, 0)
        return pltpu.make_async_copy(
            tiles_hbm.at[pl.ds(0, CHUNK)], buf, sems[j % 2]
        )

    def extract_chunk(j, buf):
        def group(v, _):
            off16 = off_v[pl.ds(j * CHUNK + v * 16, 16)]
            for lane in range(16):
                i = v * 16 + lane
                sub = off16[lane]
                o = j * CHUNK + i
                out_v[o, pl.ds(0, 16)] = buf[i, sub, pl.ds(0, 16)]
                out_v[o, pl.ds(16, 16)] = buf[i, sub, pl.ds(16, 16)]
            return 0

        lax.fori_loop(0, CHUNK // 16, group, 0)

    pending = start_gather(0, bufs[0])
    for j in range(NCHUNK):
        if j + 1 < NCHUNK:
            nxt = start_gather(j + 1, bufs[(j + 1) % 2])
        pending.wait()
        extract_chunk(j, bufs[j % 2])
        if j + 1 < NCHUNK:
            pending = nxt

    pltpu.sync_copy(out_v, out_hbm.at[pl.ds(base, B_PER_W)])


def kernel(t, table):
    tiles = table.reshape(NTILE, SUB, DIM)
    return _gather_kernel(t.astype(jnp.int32), tiles)


# speedup vs baseline: 2.2898x; 1.0003x over previous
import functools

import jax
import jax.numpy as jnp
from jax import lax
from jax.experimental import pallas as pl
from jax.experimental.pallas import tpu as pltpu
from jax.experimental.pallas import tpu_sc as plsc

NUM_EMB = 1000000
DIM = 32
BATCH = 16384
SUB = 8
NTILE = NUM_EMB // SUB

NUM_CORES = 2
NUM_SUBCORES = 16
NUM_WORKERS = NUM_CORES * NUM_SUBCORES
B_PER_W = BATCH // NUM_WORKERS
CHUNK = 16
NCHUNK = B_PER_W // CHUNK

_mesh = plsc.VectorSubcoreMesh(core_axis_name="c", subcore_axis_name="s")


@functools.partial(
    pl.kernel,
    mesh=_mesh,
    out_type=jax.ShapeDtypeStruct((BATCH, DIM), jnp.float32),
    scratch_types=[
        pltpu.VMEM((B_PER_W,), jnp.int32),
        pltpu.VMEM((B_PER_W,), jnp.int32),
        pltpu.VMEM((CHUNK, SUB, DIM), jnp.float32),
        pltpu.VMEM((CHUNK, SUB, DIM), jnp.float32),
        pltpu.VMEM((B_PER_W, DIM), jnp.float32),
        pltpu.SemaphoreType.DMA,
        pltpu.SemaphoreType.DMA,
    ],
)
def _gather_kernel(
    t_hbm, tiles_hbm, out_hbm, off_v, row_v, wide_a, wide_b, out_v, sem, sem2
):
    wid = lax.axis_index("s") * NUM_CORES + lax.axis_index("c")
    base = wid * B_PER_W
    pltpu.async_copy(t_hbm.at[pl.ds(base, B_PER_W)], row_v, sem).wait()

    def widen(v, _):
        m16 = row_v[pl.ds(v * 16, 16)]
        off_v[pl.ds(v * 16, 16)] = m16 & (SUB - 1)
        row_v[pl.ds(v * 16, 16)] = lax.shift_right_logical(m16, 3)
        return 0

    lax.fori_loop(0, B_PER_W // 16, widen, 0)

    bufs = [wide_a, wide_b]
    sems = [sem, sem2]

    def start_gather(j, buf):
        t16 = row_v[pl.ds(j * CHUNK, 16)]
        for lane in range(16):
            pltpu.async_copy(tiles_hbm.at[t16[lane]], buf.at[lane], sems[j % 2])
        return pltpu.make_async_copy(
            tiles_hbm.at[pl.ds(0, CHUNK)], buf, sems[j % 2]
        )

    def extract_chunk(j, buf):
        off16 = off_v[pl.ds(j * CHUNK, 16)]
        for lane in range(16):
            sub = off16[lane]
            o = j * CHUNK + lane
            out_v[o, pl.ds(0, 16)] = buf[lane, sub, pl.ds(0, 16)]
            out_v[o, pl.ds(16, 16)] = buf[lane, sub, pl.ds(16, 16)]

    pending = start_gather(0, bufs[0])
    for j in range(NCHUNK):
        if j + 1 < NCHUNK:
            nxt = start_gather(j + 1, bufs[(j + 1) % 2])
        pending.wait()
        extract_chunk(j, bufs[j % 2])
        if j + 1 < NCHUNK:
            pending = nxt

    pltpu.sync_copy(out_v, out_hbm.at[pl.ds(base, B_PER_W)])


def kernel(t, table):
    tiles = table.reshape(NTILE, SUB, DIM)
    return _gather_kernel(t.astype(jnp.int32), tiles)
